# baseline (speedup 1.0000x reference)
import functools

import jax
import jax.numpy as jnp
from jax import lax
from jax.experimental import pallas as pl
from jax.experimental.pallas import tpu as pltpu
from jax.experimental.pallas import tpu_sc as plsc

N = 10000
E = 320000
D = 128

EROWS = E // 128
NC = 2
NS = 16
NW = NC * NS

EH_BASE = 156
EA_BASE = 78
HR = 313

_MESH = plsc.VectorSubcoreMesh(core_axis_name="c", subcore_axis_name="s",
                               num_cores=NC, num_subcores=NS)
_SC_PARAMS = pltpu.CompilerParams(use_tc_tiling_on_sc=False)


def _deg_scale_body(row2d, h_in, hs_out, deg_sh, idxb, onesb, zb, hbuf, invb,
                    hsem, lsem):
    s = lax.axis_index("s")
    c = lax.axis_index("c")
    w = c * NS + s

    e_start = EH_BASE * s + jnp.maximum(s - 12, 0)
    e_cnt = EH_BASE + (s >= 12).astype(jnp.int32)
    r_start = 312 * w + jnp.maximum(w - 16, 0)

    zeros16 = jnp.zeros((16,), jnp.float32)
    ones16 = jnp.ones((16,), jnp.float32)

    pltpu.async_copy(h_in.at[pl.ds(r_start, HR)], hbuf, lsem)

    def fill(i, carry):
        onesb[i] = ones16
        for k in range(5):
            zb[i + 128 * k] = zeros16
        return carry
    lax.fori_loop(0, 128, fill, 0)

    pltpu.sync_copy(zb.at[pl.ds(0, 625)], deg_sh.at[pl.ds(s * 625, 625)])

    pltpu.sync_copy(row2d.at[pl.ds(e_start, EH_BASE + 1)], idxb)

    plsc.subcore_barrier()

    def hfire(i, carry):
        pltpu.async_copy(onesb, deg_sh.at[idxb.at[i]], hsem, add=True)
        return carry
    lax.fori_loop(0, e_cnt, hfire, 0)

    def hdrain(i, carry):
        pltpu.make_async_copy(onesb, deg_sh.at[idxb.at[i]], hsem).wait()
        return carry
    lax.fori_loop(0, e_cnt, hdrain, 0)

    plsc.subcore_barrier()

    pltpu.sync_copy(deg_sh.at[pl.ds(r_start, HR)], invb)

    pltpu.make_async_copy(h_in.at[pl.ds(r_start, HR)], hbuf, lsem).wait()

    def sloop(r, carry):
        s16 = 1.0 / (invb[r] + 1.0)
        for k in range(8):
            hbuf[r, pl.ds(k * 16, 16)] = hbuf[r, pl.ds(k * 16, 16)] * s16
        return carry
    lax.fori_loop(0, HR, sloop, 0)

    pltpu.sync_copy(hbuf, hs_out.at[pl.ds(r_start, HR)])


def _aggregate_body(row2d, col2d, hs_in, p_out, p_sh, ridx, cidx, msgs, sem,
                    ssem):
    s = lax.axis_index("s")
    c = lax.axis_index("c")
    w = c * NS + s

    e_start = EA_BASE * w + jnp.maximum(w - 28, 0)
    e_cnt = EA_BASE + (w >= 28).astype(jnp.int32)

    zeros16 = jnp.zeros((16,), jnp.float32)

    def zl(i, carry):
        for k in range(8):
            msgs[i, pl.ds(k * 16, 16)] = zeros16
        return carry
    lax.fori_loop(0, 128, zl, 0)
    for j in range(4):
        pltpu.sync_copy(msgs.at[pl.ds(0, 128)],
                        p_sh.at[pl.ds(s * 625 + j * 128, 128)])
    pltpu.sync_copy(msgs.at[pl.ds(0, 113)],
                    p_sh.at[pl.ds(s * 625 + 512, 113)])

    plsc.subcore_barrier()

    def run_half(sz):
        pltpu.async_copy(hs_in.at[ridx.at[0]], msgs.at[pl.ds(0, 128)], sem)

        def ml(j, carry):
            off = (j % 2) * 128
            cur = msgs.at[pl.ds(off, 128)]
            pltpu.make_async_copy(hs_in.at[ridx.at[j]], cur, sem).wait()

            @pl.when(j > 0)
            def _drain_prev():
                poff = ((j - 1) % 2) * 128
                pltpu.make_async_copy(msgs.at[pl.ds(poff, 128)],
                                      p_sh.at[cidx.at[j - 1]], ssem).wait()

            @pl.when(j < sz - 1)
            def _prefetch():
                noff = ((j + 1) % 2) * 128
                pltpu.async_copy(hs_in.at[ridx.at[j + 1]],
                                 msgs.at[pl.ds(noff, 128)], sem)

            pltpu.async_copy(cur, p_sh.at[cidx.at[j]], ssem, add=True)
            return carry
        lax.fori_loop(0, sz, ml, 0)
        loff = ((sz - 1) % 2) * 128
        pltpu.make_async_copy(msgs.at[pl.ds(loff, 128)],
                              p_sh.at[cidx.at[sz - 1]], ssem).wait()

    pltpu.sync_copy(row2d.at[pl.ds(e_start, 39)], ridx.at[pl.ds(0, 39)])
    pltpu.sync_copy(col2d.at[pl.ds(e_start, 39)], cidx.at[pl.ds(0, 39)])
    run_half(39)
    pltpu.sync_copy(row2d.at[pl.ds(e_start + 39, 40)], ridx)
    pltpu.sync_copy(col2d.at[pl.ds(e_start + 39, 40)], cidx)
    run_half(e_cnt - 39)

    plsc.subcore_barrier()

    pltpu.sync_copy(p_sh.at[pl.ds(s * 625, 625)],
                    p_out.at[c, pl.ds(s * 625, 625)])


_deg_scale = functools.partial(
    pl.kernel,
    out_type=jax.ShapeDtypeStruct((N, D), jnp.float32),
    mesh=_MESH,
    scratch_types=[
        pltpu.VMEM_SHARED((N, 16), jnp.float32),
        pltpu.VMEM((EH_BASE + 1, 128), jnp.int32),
        pltpu.VMEM((128, 16), jnp.float32),
        pltpu.VMEM((640, 16), jnp.float32),
        pltpu.VMEM((HR, 128), jnp.float32),
        pltpu.VMEM((HR, 16), jnp.float32),
        pltpu.SemaphoreType.DMA,
        pltpu.SemaphoreType.DMA,
    ],
    compiler_params=_SC_PARAMS,
)(_deg_scale_body)


_aggregate = functools.partial(
    pl.kernel,
    out_type=jax.ShapeDtypeStruct((NC, N, D), jnp.float32),
    mesh=_MESH,
    scratch_types=[
        pltpu.VMEM_SHARED((N, D), jnp.float32),
        pltpu.VMEM((40, 128), jnp.int32),
        pltpu.VMEM((40, 128), jnp.int32),
        pltpu.VMEM((256, 128), jnp.float32),
        pltpu.SemaphoreType.DMA,
        pltpu.SemaphoreType.DMA,
    ],
    compiler_params=_SC_PARAMS,
)(_aggregate_body)


def _dense_body(p0, p1, hs, w_ref, b_ref, g_ref, be_ref, o_ref):
    agg = p0[...] + p1[...] + hs[...]
    lin = lax.dot_general(agg, w_ref[...], (((1,), (1,)), ((), ())),
                          preferred_element_type=jnp.float32) + b_ref[...]
    h = jnp.maximum(lin, 0.0)
    mean = jnp.mean(h, axis=-1, keepdims=True)
    var = jnp.mean((h - mean) ** 2, axis=-1, keepdims=True)
    o_ref[...] = (h - mean) * lax.rsqrt(var + 1e-5) * g_ref[...] + be_ref[...]


_BLK = 400


def _dense(p0, p1, hs, W, b, gamma, beta):
    blk = pl.BlockSpec((_BLK, D), lambda i: (i, 0))
    full = pl.BlockSpec((D, D), lambda i: (0, 0))
    vec = pl.BlockSpec((1, D), lambda i: (0, 0))
    return pl.pallas_call(
        _dense_body,
        grid=(N // _BLK,),
        in_specs=[blk, blk, blk, full, vec, vec, vec],
        out_specs=blk,
        out_shape=jax.ShapeDtypeStruct((N, D), jnp.float32),
    )(p0, p1, hs, W, b, gamma, beta)


def kernel(H, edge_index, num_nodes, W, b, gamma, beta):
    del num_nodes
    row2d = edge_index[0].reshape(EROWS, 128)
    col2d = edge_index[1].reshape(EROWS, 128)

    hs = _deg_scale(row2d, H)
    parts = _aggregate(row2d, col2d, hs)
    return _dense(parts[0], parts[1], hs, W,
                  b.reshape(1, D), gamma.reshape(1, D), beta.reshape(1, D))


# speedup vs baseline: 1.1360x; 1.1360x over previous
import functools

import jax
import jax.numpy as jnp
from jax import lax
from jax.experimental import pallas as pl
from jax.experimental.pallas import tpu as pltpu
from jax.experimental.pallas import tpu_sc as plsc

N = 10000
E = 320000
D = 128

EROWS = E // 128
NC = 2
NS = 16
NW = NC * NS

EH_BASE = 156
EA_BASE = 78
HR = 313

_MESH = plsc.VectorSubcoreMesh(core_axis_name="c", subcore_axis_name="s",
                               num_cores=NC, num_subcores=NS)
_SC_PARAMS = pltpu.CompilerParams(use_tc_tiling_on_sc=False)


def _deg_scale_body(e3, h_in, hs_out, deg_sh, idxb, onesb, zb, hbuf, invb,
                    hsem, lsem):
    s = lax.axis_index("s")
    c = lax.axis_index("c")
    w = c * NS + s

    e_start = EH_BASE * s + jnp.maximum(s - 12, 0)
    e_cnt = EH_BASE + (s >= 12).astype(jnp.int32)
    r_start = 312 * w + jnp.maximum(w - 16, 0)

    zeros16 = jnp.zeros((16,), jnp.float32)
    ones16 = jnp.ones((16,), jnp.float32)

    pltpu.async_copy(h_in.at[pl.ds(r_start, HR)], hbuf, lsem)

    def fill(i, carry):
        onesb[i] = ones16
        for k in range(5):
            zb[i + 128 * k] = zeros16
        return carry
    lax.fori_loop(0, 128, fill, 0)

    pltpu.sync_copy(zb.at[pl.ds(0, 625)], deg_sh.at[pl.ds(s * 625, 625)])

    pltpu.sync_copy(e3.at[0, pl.ds(e_start, EH_BASE + 1)], idxb)

    plsc.subcore_barrier()

    def hfire(i, carry):
        pltpu.async_copy(onesb, deg_sh.at[idxb.at[i]], hsem, add=True)
        return carry
    lax.fori_loop(0, e_cnt, hfire, 0)

    def hdrain(i, carry):
        pltpu.make_async_copy(onesb, deg_sh.at[idxb.at[i]], hsem).wait()
        return carry
    lax.fori_loop(0, e_cnt, hdrain, 0)

    plsc.subcore_barrier()

    pltpu.sync_copy(deg_sh.at[pl.ds(r_start, HR)], invb)

    pltpu.make_async_copy(h_in.at[pl.ds(r_start, HR)], hbuf, lsem).wait()

    def sloop(r, carry):
        s16 = 1.0 / (invb[r] + 1.0)
        for k in range(8):
            hbuf[r, pl.ds(k * 16, 16)] = hbuf[r, pl.ds(k * 16, 16)] * s16
        return carry
    lax.fori_loop(0, HR, sloop, 0)

    pltpu.sync_copy(hbuf, hs_out.at[pl.ds(r_start, HR)])


def _aggregate_body(e3, hs_in, p_out, p_sh, ridx, cidx, msgs, sem, ssem):
    s = lax.axis_index("s")
    c = lax.axis_index("c")
    w = c * NS + s

    e_start = EA_BASE * w + jnp.maximum(w - 28, 0)
    e_cnt = EA_BASE + (w >= 28).astype(jnp.int32)

    zeros16 = jnp.zeros((16,), jnp.float32)

    def zl(i, carry):
        for k in range(8):
            msgs[i, pl.ds(k * 16, 16)] = zeros16
        return carry
    lax.fori_loop(0, 128, zl, 0)
    for j in range(4):
        pltpu.sync_copy(msgs.at[pl.ds(0, 128)],
                        p_sh.at[pl.ds(s * 625 + j * 128, 128)])
    pltpu.sync_copy(msgs.at[pl.ds(0, 113)],
                    p_sh.at[pl.ds(s * 625 + 512, 113)])

    plsc.subcore_barrier()

    def run_half(sz):
        pltpu.async_copy(hs_in.at[ridx.at[0]], msgs.at[pl.ds(0, 128)], sem)

        def ml(j, carry):
            off = (j % 2) * 128
            cur = msgs.at[pl.ds(off, 128)]
            pltpu.make_async_copy(hs_in.at[ridx.at[j]], cur, sem).wait()

            @pl.when(j > 0)
            def _drain_prev():
                poff = ((j - 1) % 2) * 128
                pltpu.make_async_copy(msgs.at[pl.ds(poff, 128)],
                                      p_sh.at[cidx.at[j - 1]], ssem).wait()

            @pl.when(j < sz - 1)
            def _prefetch():
                noff = ((j + 1) % 2) * 128
                pltpu.async_copy(hs_in.at[ridx.at[j + 1]],
                                 msgs.at[pl.ds(noff, 128)], sem)

            pltpu.async_copy(cur, p_sh.at[cidx.at[j]], ssem, add=True)
            return carry
        lax.fori_loop(0, sz, ml, 0)
        loff = ((sz - 1) % 2) * 128
        pltpu.make_async_copy(msgs.at[pl.ds(loff, 128)],
                              p_sh.at[cidx.at[sz - 1]], ssem).wait()

    pltpu.sync_copy(e3.at[0, pl.ds(e_start, 39)], ridx.at[pl.ds(0, 39)])
    pltpu.sync_copy(e3.at[1, pl.ds(e_start, 39)], cidx.at[pl.ds(0, 39)])
    run_half(39)
    pltpu.sync_copy(e3.at[0, pl.ds(e_start + 39, 40)], ridx)
    pltpu.sync_copy(e3.at[1, pl.ds(e_start + 39, 40)], cidx)
    run_half(e_cnt - 39)

    plsc.subcore_barrier()

    pltpu.sync_copy(p_sh.at[pl.ds(s * 625, 625)],
                    p_out.at[c, pl.ds(s * 625, 625)])


_deg_scale = functools.partial(
    pl.kernel,
    out_type=jax.ShapeDtypeStruct((N, D), jnp.float32),
    mesh=_MESH,
    scratch_types=[
        pltpu.VMEM_SHARED((N, 16), jnp.float32),
        pltpu.VMEM((EH_BASE + 1, 128), jnp.int32),
        pltpu.VMEM((128, 16), jnp.float32),
        pltpu.VMEM((640, 16), jnp.float32),
        pltpu.VMEM((HR, 128), jnp.float32),
        pltpu.VMEM((HR, 16), jnp.float32),
        pltpu.SemaphoreType.DMA,
        pltpu.SemaphoreType.DMA,
    ],
    compiler_params=_SC_PARAMS,
)(_deg_scale_body)


_aggregate = functools.partial(
    pl.kernel,
    out_type=jax.ShapeDtypeStruct((NC, N, D), jnp.float32),
    mesh=_MESH,
    scratch_types=[
        pltpu.VMEM_SHARED((N, D), jnp.float32),
        pltpu.VMEM((40, 128), jnp.int32),
        pltpu.VMEM((40, 128), jnp.int32),
        pltpu.VMEM((256, 128), jnp.float32),
        pltpu.SemaphoreType.DMA,
        pltpu.SemaphoreType.DMA,
    ],
    compiler_params=_SC_PARAMS,
)(_aggregate_body)


def _dense_body(p, hs, w_ref, b_ref, g_ref, be_ref, o_ref):
    agg = p[0] + p[1] + hs[...]
    lin = lax.dot_general(agg, w_ref[...], (((1,), (1,)), ((), ())),
                          preferred_element_type=jnp.float32) + b_ref[...]
    h = jnp.maximum(lin, 0.0)
    mean = jnp.mean(h, axis=-1, keepdims=True)
    var = jnp.mean((h - mean) ** 2, axis=-1, keepdims=True)
    o_ref[...] = (h - mean) * lax.rsqrt(var + 1e-5) * g_ref[...] + be_ref[...]


_BLK = 1000


def _dense(p, hs, W, b, gamma, beta):
    pblk = pl.BlockSpec((NC, _BLK, D), lambda i: (0, i, 0))
    blk = pl.BlockSpec((_BLK, D), lambda i: (i, 0))
    full = pl.BlockSpec((D, D), lambda i: (0, 0))
    vec = pl.BlockSpec((1, D), lambda i: (0, 0))
    return pl.pallas_call(
        _dense_body,
        grid=(N // _BLK,),
        in_specs=[pblk, blk, full, vec, vec, vec],
        out_specs=blk,
        out_shape=jax.ShapeDtypeStruct((N, D), jnp.float32),
    )(p, hs, W, b, gamma, beta)


def kernel(H, edge_index, num_nodes, W, b, gamma, beta):
    del num_nodes
    e3 = edge_index.reshape(2, EROWS, 128)

    hs = _deg_scale(e3, H)
    parts = _aggregate(e3, hs)
    return _dense(parts, hs, W,
                  b.reshape(1, D), gamma.reshape(1, D), beta.reshape(1, D))


# speedup vs baseline: 1.1513x; 1.0134x over previous
import functools

import jax
import jax.numpy as jnp
from jax import lax
from jax.experimental import pallas as pl
from jax.experimental.pallas import tpu as pltpu
from jax.experimental.pallas import tpu_sc as plsc

N = 10000
E = 320000
D = 128

EROWS = E // 128
NC = 2
NS = 16
NW = NC * NS

EH_BASE = 156
EA_BASE = 78
HR = 313

_MESH = plsc.VectorSubcoreMesh(core_axis_name="c", subcore_axis_name="s",
                               num_cores=NC, num_subcores=NS)
_SC_PARAMS = pltpu.CompilerParams(use_tc_tiling_on_sc=False)


def _deg_scale_body(e3, h_in, hs_out, deg_sh, idxb, onesb, zb, hbuf, invb,
                    hsem, lsem):
    s = lax.axis_index("s")
    c = lax.axis_index("c")
    w = c * NS + s

    e_start = EH_BASE * s + jnp.maximum(s - 12, 0)
    e_cnt = EH_BASE + (s >= 12).astype(jnp.int32)
    r_start = 312 * w + jnp.maximum(w - 16, 0)

    zeros16 = jnp.zeros((16,), jnp.float32)
    ones16 = jnp.ones((16,), jnp.float32)

    pltpu.async_copy(h_in.at[pl.ds(r_start, HR)], hbuf, lsem)

    def fill(i, carry):
        onesb[i] = ones16
        for k in range(5):
            zb[i + 128 * k] = zeros16
        return carry
    lax.fori_loop(0, 128, fill, 0)

    pltpu.sync_copy(zb.at[pl.ds(0, 625)], deg_sh.at[pl.ds(s * 625, 625)])

    pltpu.sync_copy(e3.at[0, pl.ds(e_start, EH_BASE + 1)], idxb)

    plsc.subcore_barrier()

    def hfire(i, carry):
        pltpu.async_copy(onesb, deg_sh.at[idxb.at[i]], hsem, add=True)
        return carry
    lax.fori_loop(0, e_cnt, hfire, 0)

    def hdrain(i, carry):
        pltpu.make_async_copy(onesb, deg_sh.at[idxb.at[i]], hsem).wait()
        return carry
    lax.fori_loop(0, e_cnt, hdrain, 0)

    plsc.subcore_barrier()

    pltpu.sync_copy(deg_sh.at[pl.ds(r_start, HR)], invb)

    pltpu.make_async_copy(h_in.at[pl.ds(r_start, HR)], hbuf, lsem).wait()

    def sloop(r, carry):
        s16 = 1.0 / (invb[r] + 1.0)
        for k in range(8):
            hbuf[r, pl.ds(k * 16, 16)] = hbuf[r, pl.ds(k * 16, 16)] * s16
        return carry
    lax.fori_loop(0, HR, sloop, 0)

    pltpu.sync_copy(hbuf, hs_out.at[pl.ds(r_start, HR)])


def _aggregate_body(e3, hs_in, p_out, p_sh, ridx, cidx, msgs, sem, ssem):
    s = lax.axis_index("s")
    c = lax.axis_index("c")
    w = c * NS + s

    e_start = EA_BASE * w + jnp.maximum(w - 28, 0)
    e_cnt = EA_BASE + (w >= 28).astype(jnp.int32)

    zeros16 = jnp.zeros((16,), jnp.float32)

    def zl(i, carry):
        for k in range(8):
            msgs[i, pl.ds(k * 16, 16)] = zeros16
        return carry
    lax.fori_loop(0, 128, zl, 0)
    for j in range(4):
        pltpu.sync_copy(msgs.at[pl.ds(0, 128)],
                        p_sh.at[pl.ds(s * 625 + j * 128, 128)])
    pltpu.sync_copy(msgs.at[pl.ds(0, 113)],
                    p_sh.at[pl.ds(s * 625 + 512, 113)])

    plsc.subcore_barrier()

    def run_half(sz):
        pltpu.async_copy(hs_in.at[ridx.at[0]], msgs.at[pl.ds(0, 128)], sem)

        def ml(j, carry):
            off = (j % 2) * 128
            cur = msgs.at[pl.ds(off, 128)]
            pltpu.make_async_copy(hs_in.at[ridx.at[j]], cur, sem).wait()

            @pl.when(j > 0)
            def _drain_prev():
                poff = ((j - 1) % 2) * 128
                pltpu.make_async_copy(msgs.at[pl.ds(poff, 128)],
                                      p_sh.at[cidx.at[j - 1]], ssem).wait()

            @pl.when(j < sz - 1)
            def _prefetch():
                noff = ((j + 1) % 2) * 128
                pltpu.async_copy(hs_in.at[ridx.at[j + 1]],
                                 msgs.at[pl.ds(noff, 128)], sem)

            pltpu.async_copy(cur, p_sh.at[cidx.at[j]], ssem, add=True)
            return carry
        lax.fori_loop(0, sz, ml, 0)
        loff = ((sz - 1) % 2) * 128
        pltpu.make_async_copy(msgs.at[pl.ds(loff, 128)],
                              p_sh.at[cidx.at[sz - 1]], ssem).wait()

    pltpu.sync_copy(e3.at[0, pl.ds(e_start, 39)], ridx.at[pl.ds(0, 39)])
    pltpu.sync_copy(e3.at[1, pl.ds(e_start, 39)], cidx.at[pl.ds(0, 39)])
    run_half(39)
    pltpu.sync_copy(e3.at[0, pl.ds(e_start + 39, 40)], ridx)
    pltpu.sync_copy(e3.at[1, pl.ds(e_start + 39, 40)], cidx)
    run_half(e_cnt - 39)

    plsc.subcore_barrier()

    pltpu.sync_copy(p_sh.at[pl.ds(s * 625, 625)],
                    p_out.at[c, pl.ds(s * 625, 625)])


_deg_scale = functools.partial(
    pl.kernel,
    out_type=jax.ShapeDtypeStruct((N, D), jnp.float32),
    mesh=_MESH,
    scratch_types=[
        pltpu.VMEM_SHARED((N, 16), jnp.float32),
        pltpu.VMEM((EH_BASE + 1, 128), jnp.int32),
        pltpu.VMEM((128, 16), jnp.float32),
        pltpu.VMEM((640, 16), jnp.float32),
        pltpu.VMEM((HR, 128), jnp.float32),
        pltpu.VMEM((HR, 16), jnp.float32),
        pltpu.SemaphoreType.DMA,
        pltpu.SemaphoreType.DMA,
    ],
    compiler_params=_SC_PARAMS,
)(_deg_scale_body)


_aggregate = functools.partial(
    pl.kernel,
    out_type=jax.ShapeDtypeStruct((NC, N, D), jnp.float32),
    mesh=_MESH,
    scratch_types=[
        pltpu.VMEM_SHARED((N, D), jnp.float32),
        pltpu.VMEM((40, 128), jnp.int32),
        pltpu.VMEM((40, 128), jnp.int32),
        pltpu.VMEM((256, 128), jnp.float32),
        pltpu.SemaphoreType.DMA,
        pltpu.SemaphoreType.DMA,
    ],
    compiler_params=_SC_PARAMS,
)(_aggregate_body)


def _dense_body(p, hs, w_ref, b_ref, g_ref, be_ref, o_ref):
    agg = p[0] + p[1] + hs[...]
    lin = lax.dot_general(agg, w_ref[...], (((1,), (1,)), ((), ())),
                          preferred_element_type=jnp.float32) + b_ref[...]
    h = jnp.maximum(lin, 0.0)
    mean = jnp.mean(h, axis=-1, keepdims=True)
    var = jnp.mean((h - mean) ** 2, axis=-1, keepdims=True)
    o_ref[...] = (h - mean) * lax.rsqrt(var + 1e-5) * g_ref[...] + be_ref[...]


_BLK = 2000


def _dense(p, hs, W, b, gamma, beta):
    pblk = pl.BlockSpec((NC, _BLK, D), lambda i: (0, i, 0))
    blk = pl.BlockSpec((_BLK, D), lambda i: (i, 0))
    full = pl.BlockSpec((D, D), lambda i: (0, 0))
    vec = pl.BlockSpec((1, D), lambda i: (0, 0))
    return pl.pallas_call(
        _dense_body,
        grid=(N // _BLK,),
        in_specs=[pblk, blk, full, vec, vec, vec],
        out_specs=blk,
        out_shape=jax.ShapeDtypeStruct((N, D), jnp.float32),
    )(p, hs, W, b, gamma, beta)


def kernel(H, edge_index, num_nodes, W, b, gamma, beta):
    del num_nodes
    e3 = edge_index.reshape(2, EROWS, 128)

    hs = _deg_scale(e3, H)
    parts = _aggregate(e3, hs)
    return _dense(parts, hs, W,
                  b.reshape(1, D), gamma.reshape(1, D), beta.reshape(1, D))
